# Initial kernel scaffold; baseline (speedup 1.0000x reference)
import jax
import jax.numpy as jnp
from jax.experimental import pallas as pl


def kernel(news_representation, category, subCategory, category_table, subCategory_table):
    raise NotImplementedError("write your pallas kernel here")


# speedup vs baseline: 1.0708x; 1.0708x over previous
import functools

import jax
import jax.numpy as jnp
from jax import lax
from jax.experimental import pallas as pl
from jax.experimental.pallas import tpu as pltpu
from jax.experimental.pallas import tpu_sc as plsc

_B = 4096
_L = 50
_D_NEWS = 400
_CAT_NUM = 20
_SUBCAT_NUM = 300
_CAT_DIM = 100
_SUBCAT_DIM = 100
_D_EMB = _CAT_DIM + _SUBCAT_DIM
_D_GATHER = 256
_D_OUT = _D_NEWS + _D_EMB
_N = _B * _L
_N_FUSED = _CAT_NUM * _SUBCAT_NUM

_NUM_CORES = 2
_NUM_SUBCORES = 16
_NW = _NUM_CORES * _NUM_SUBCORES
_ROWS_PER_W = _N // _NW
_CHUNK = 64
_NCHUNK = _ROWS_PER_W // _CHUNK
_LANES = 16

_ROW_BLOCK = 1024


def _fuse_tables_tc(cat_tab, sub_tab):
    def body(cat_ref, sub_ref, out_ref):
        cat = cat_ref[...]
        sub = sub_ref[...]
        cat_rep = lax.broadcast_in_dim(
            cat, (_CAT_NUM, _SUBCAT_NUM, _CAT_DIM), (0, 2)
        ).reshape(_N_FUSED, _CAT_DIM)
        sub_rep = lax.broadcast_in_dim(
            sub, (_CAT_NUM, _SUBCAT_NUM, _SUBCAT_DIM), (1, 2)
        ).reshape(_N_FUSED, _SUBCAT_DIM)
        pad = jnp.zeros((_N_FUSED, _D_GATHER - _D_EMB), jnp.float32)
        out_ref[...] = jnp.concatenate([cat_rep, sub_rep, pad], axis=1)

    return pl.pallas_call(
        body,
        out_shape=jax.ShapeDtypeStruct((_N_FUSED, _D_GATHER), jnp.float32),
    )(cat_tab, sub_tab)


def _make_sc_gather():
    mesh = plsc.VectorSubcoreMesh(core_axis_name="c", subcore_axis_name="s")

    @functools.partial(
        pl.kernel,
        mesh=mesh,
        out_type=jax.ShapeDtypeStruct((_N, _D_GATHER), jnp.float32),
        scratch_types=[
            pltpu.VMEM((_CHUNK,), jnp.int32),
            pltpu.VMEM((_CHUNK,), jnp.int32),
            pltpu.VMEM((_CHUNK,), jnp.int32),
            pltpu.VMEM((_CHUNK, _D_GATHER), jnp.float32),
            pltpu.SemaphoreType.DMA,
        ],
    )
    def sc_gather(cat_hbm, sub_hbm, fused_tab_hbm, emb_hbm,
                  cat_idx_v, sub_idx_v, fused_idx_v, gbuf_v, sem_g):
        wid = lax.axis_index("s") * _NUM_CORES + lax.axis_index("c")
        base0 = wid * _ROWS_PER_W

        def step(i, carry):
            base = pl.multiple_of(base0 + i * _CHUNK, _CHUNK)
            pltpu.sync_copy(cat_hbm.at[pl.ds(base, _CHUNK)], cat_idx_v)
            pltpu.sync_copy(sub_hbm.at[pl.ds(base, _CHUNK)], sub_idx_v)
            for j in range(_CHUNK // _LANES):
                sl = pl.ds(j * _LANES, _LANES)
                fused_idx_v[sl] = cat_idx_v[sl] * _SUBCAT_NUM + sub_idx_v[sl]
            pltpu.async_copy(
                fused_tab_hbm.at[fused_idx_v], gbuf_v, sem_g).wait()
            pltpu.sync_copy(gbuf_v, emb_hbm.at[pl.ds(base, _CHUNK)])
            return carry

        lax.fori_loop(0, _NCHUNK, step, 0)

    return sc_gather


_SC_GATHER = _make_sc_gather()


def _concat_tc(news2d, emb):
    def body(news_ref, emb_ref, out_ref):
        out_ref[...] = jnp.concatenate(
            [news_ref[...], emb_ref[:, : _D_EMB]], axis=1)

    grid = (_N // _ROW_BLOCK,)
    return pl.pallas_call(
        body,
        grid=grid,
        in_specs=[
            pl.BlockSpec((_ROW_BLOCK, _D_NEWS), lambda i: (i, 0)),
            pl.BlockSpec((_ROW_BLOCK, _D_GATHER), lambda i: (i, 0)),
        ],
        out_specs=pl.BlockSpec((_ROW_BLOCK, _D_OUT), lambda i: (i, 0)),
        out_shape=jax.ShapeDtypeStruct((_N, _D_OUT), jnp.float32),
    )(news2d, emb)


def kernel(news_representation, category, subCategory, category_table,
           subCategory_table):
    news2d = news_representation.reshape(_N, _D_NEWS)
    cat1d = category.reshape(_N).astype(jnp.int32)
    sub1d = subCategory.reshape(_N).astype(jnp.int32)
    fused_tab = _fuse_tables_tc(category_table, subCategory_table)
    emb = _SC_GATHER(cat1d, sub1d, fused_tab)
    out = _concat_tc(news2d, emb)
    return out.reshape(_B, _L, _D_OUT)
